# Initial kernel scaffold; baseline (speedup 1.0000x reference)
import jax
import jax.numpy as jnp
from jax.experimental import pallas as pl


def kernel(embeddings, flatten_idx):
    raise NotImplementedError("write your pallas kernel here")


# speedup vs baseline: 94.6021x; 94.6021x over previous
import functools

import jax
import jax.numpy as jnp
from jax import lax
from jax.experimental import pallas as pl
from jax.experimental.pallas import tpu as pltpu
from jax.experimental.pallas import tpu_sc as plsc

N = 320000
H = 128
G = 10000
NW = 32
C = N // NW
B = 512
ROWW = 272
NG = C // 16


def _rsqrt_s(s):
    x = jnp.maximum(s, 1e-30)
    i = lax.bitcast_convert_type(x, jnp.int32)
    i = jnp.int32(0x5F3759DF) - (i >> 1)
    y = lax.bitcast_convert_type(i, jnp.float32)
    hx = x * 0.5
    for _ in range(2):
        y = y * (1.5 - hx * y * y)
    return jnp.minimum(y, 1e8)


def _sc_body(emb_hbm, idx_hbm, acc_hbm,
             idx_v, prev8_v, bounds_v, ext_v, rows_v, stage_a, stage_b,
             sem_a, sem_b, blk_s):
    wid = lax.axis_index("s") * 2 + lax.axis_index("c")
    w0 = pl.multiple_of(wid * C, 8)

    pltpu.sync_copy(idx_hbm.at[pl.ds(w0, C)], idx_v.at[pl.ds(8, C)])
    pltpu.sync_copy(idx_hbm.at[pl.ds(pl.multiple_of(jnp.maximum(w0 - 8, 0), 8), 8)],
                    prev8_v.at[pl.ds(0, 8)])
    prev = jnp.where(wid > 0, prev8_v[...][7], jnp.int32(-1))
    head = idx_v[pl.ds(0, 16)]
    lane = lax.iota(jnp.int32, 16)
    idx_v[pl.ds(0, 16)] = jnp.where(lane == 7, prev, head)

    def bscan(g, off):
        v = idx_v[pl.ds(8 + 16 * g, 16)]
        pv = idx_v[pl.ds(7 + 16 * g, 16)]
        msk = v != pv
        pos = lax.iota(jnp.int32, 16) + (w0 + 16 * g)
        mi = msk.astype(jnp.int32)
        excl = plsc.cumsum(mi) - mi
        dest = jnp.where(msk, off + excl, jnp.int32(C + 8))
        plsc.store_scatter(bounds_v, [dest], pos)
        cnt = jnp.max(plsc.all_reduce_population_count(msk))
        return off + cnt

    nseg = lax.fori_loop(0, NG, bscan, jnp.int32(0))

    last_id = idx_v[pl.ds(7 + C, 16)][0]

    def ecma_cond(st):
        m, t = st
        return jnp.logical_and(t < 0, m < N)

    def ecma_body(st):
        m, _ = st
        m_eff = pl.multiple_of(jnp.minimum(m, N - 16), 8)
        pltpu.sync_copy(idx_hbm.at[pl.ds(m_eff, 16)], ext_v)
        msk = ext_v[...] != last_id
        has = jnp.max(plsc.all_reduce_population_count(msk)) > 0
        ffs = jnp.max(plsc.all_reduce_ffs(msk))
        t_new = jnp.where(has, m_eff + ffs, jnp.int32(-1))
        return m + 16, t_new

    _, t_end = lax.while_loop(ecma_cond, ecma_body,
                              (w0 + jnp.int32(C), jnp.int32(-1)))
    t_end = jnp.where(t_end < 0, jnp.int32(N), t_end)
    bounds_v[pl.ds(nseg, 16)] = jnp.full((16,), t_end, jnp.int32)
    blk_s[0] = jnp.int32(-1)
    blk_s[1] = jnp.int32(0)

    zero16 = jnp.zeros((16,), jnp.float32)
    zaccs = (zero16,) * 16

    def flush(fc, sid, count, accs):
        def emit(stage, sem):
            @pl.when(fc >= 2)
            def _():
                pltpu.make_async_copy(stage, acc_hbm.at[sid], sem).wait()

            for jj in range(8):
                stage[pl.ds(16 * jj, 16)] = accs[jj]
                stage[pl.ds(128 + 16 * jj, 16)] = accs[8 + jj]
            stage[pl.ds(256, 16)] = jnp.full((16,), count.astype(jnp.float32))
            pltpu.make_async_copy(stage, acc_hbm.at[sid], sem).start()

        @pl.when((fc & 1) == 0)
        def _():
            emit(stage_a, sem_a)

        @pl.when((fc & 1) == 1)
        def _():
            emit(stage_b, sem_b)

    def make_row_accum(base):
        def row_accum(r, accs):
            q = r - base
            vals = [rows_v[q, pl.ds(16 * j, 16)] for j in range(8)]
            sq01 = vals[0] * vals[0] + vals[1] * vals[1]
            sq23 = vals[2] * vals[2] + vals[3] * vals[3]
            sq45 = vals[4] * vals[4] + vals[5] * vals[5]
            sq67 = vals[6] * vals[6] + vals[7] * vals[7]
            sq = (sq01 + sq23) + (sq45 + sq67)
            scale = jnp.full((16,), _rsqrt_s(jnp.sum(sq)), jnp.float32)
            new = [accs[j] + vals[j] for j in range(8)]
            new += [accs[8 + j] + vals[j] * scale for j in range(8)]
            return tuple(new)
        return row_accum

    bnd0 = bounds_v[pl.ds(0, 16)][0]
    k0 = (bnd0 - w0) // 16
    t_eff = jnp.where(nseg > 0, t_end, w0)

    def g_cond(st):
        k = st[0]
        return w0 + 16 * k < t_eff

    def g_body(st):
        k, jb, nextb, ostart, ovalid, fc, accs = st
        gs = w0 + 16 * k
        ge = jnp.minimum(gs + 16, t_eff)
        blk = (16 * k) // B

        @pl.when(blk != blk_s[0])
        def _():
            p = jnp.minimum(w0 + blk * B, N - B)
            pltpu.sync_copy(emb_hbm.at[pl.ds(p, B)], rows_v)
            blk_s[0] = blk
            blk_s[1] = p

        base = blk_s[1]
        row_accum = make_row_accum(base)
        fast = jnp.logical_and(nextb >= gs + 16, ge == gs + 16)

        def fast_fn(jb, nextb, ostart, ovalid, fc, accs):
            q0 = gs - base
            new_s = list(accs[:8])
            new_n = list(accs[8:])
            for u in range(16):
                vals = [rows_v[q0 + u, pl.ds(16 * j, 16)] for j in range(8)]
                sq01 = vals[0] * vals[0] + vals[1] * vals[1]
                sq23 = vals[2] * vals[2] + vals[3] * vals[3]
                sq45 = vals[4] * vals[4] + vals[5] * vals[5]
                sq67 = vals[6] * vals[6] + vals[7] * vals[7]
                sq = (sq01 + sq23) + (sq45 + sq67)
                scale = jnp.full((16,), _rsqrt_s(jnp.sum(sq)), jnp.float32)
                for j in range(8):
                    new_s[j] = new_s[j] + vals[j]
                    new_n[j] = new_n[j] + vals[j] * scale
            return jb, nextb, ostart, ovalid, fc, tuple(new_s + new_n)

        def slow_fn(jb, nextb, ostart, ovalid, fc, accs):
            def s_cond(st):
                return st[2] < ge

            def s_body(st):
                cur, jb, nextb, ostart, ovalid, fc, accs = st
                lo = jnp.where(ovalid > 0, cur, nextb)
                accs = lax.fori_loop(lo, nextb, row_accum, accs)

                @pl.when(ovalid > 0)
                def _():
                    sid = idx_v[pl.ds(8 + ostart - w0, 16)][0]
                    flush(fc, sid, nextb - ostart, accs)

                fc = fc + jnp.where(ovalid > 0, 1, 0)
                ostart = nextb
                jb = jb + 1
                nextb2 = bounds_v[pl.ds(jb, 16)][0]
                return nextb, jb, nextb2, ostart, jnp.int32(1), fc, zaccs

            cur, jb, nextb, ostart, ovalid, fc, accs = lax.while_loop(
                s_cond, s_body, (gs, jb, nextb, ostart, ovalid, fc, accs))
            lo = jnp.where(ovalid > 0, jnp.maximum(cur, gs), ge)
            accs = lax.fori_loop(lo, ge, row_accum, accs)
            return jb, nextb, ostart, ovalid, fc, accs

        jb, nextb, ostart, ovalid, fc, accs = lax.cond(
            fast, fast_fn, slow_fn, jb, nextb, ostart, ovalid, fc, accs)
        return (k + 1, jb, nextb, ostart, ovalid, fc, accs)

    st = lax.while_loop(
        g_cond, g_body,
        (k0, jnp.int32(0), bnd0, jnp.int32(0), jnp.int32(0), jnp.int32(0),
         zaccs))
    _, _, _, ostart, ovalid, fc, accs = st

    @pl.when(ovalid > 0)
    def _():
        sid = idx_v[pl.ds(8 + ostart - w0, 16)][0]
        flush(fc, sid, t_end - ostart, accs)

    fc = fc + jnp.where(ovalid > 0, 1, 0)

    @pl.when(fc >= 1)
    def _():
        pltpu.make_async_copy(stage_a, acc_hbm.at[0], sem_a).wait()

    @pl.when(fc >= 2)
    def _():
        pltpu.make_async_copy(stage_b, acc_hbm.at[0], sem_b).wait()


_sc_segsum = functools.partial(
    pl.kernel,
    out_type=jax.ShapeDtypeStruct((G, ROWW), jnp.float32),
    mesh=plsc.VectorSubcoreMesh(core_axis_name="c", subcore_axis_name="s",
                                num_cores=2, num_subcores=16),
    compiler_params=pltpu.CompilerParams(needs_layout_passes=False),
    scratch_types=[
        pltpu.VMEM((C + 32,), jnp.int32),
        pltpu.VMEM((16,), jnp.int32),
        pltpu.VMEM((C + 16,), jnp.int32),
        pltpu.VMEM((16,), jnp.int32),
        pltpu.VMEM((B, H), jnp.float32),
        pltpu.VMEM((ROWW,), jnp.float32),
        pltpu.VMEM((ROWW,), jnp.float32),
        pltpu.SemaphoreType.DMA,
        pltpu.SemaphoreType.DMA,
        pltpu.SMEM((2,), jnp.int32),
    ],
)(_sc_body)


def _tc_finish_body(acc_ref, avg_ref, mdis_ref):
    sums = acc_ref[:, 0:128]
    nsums = acc_ref[:, 128:256]
    cnt = acc_ref[:, 256:257]
    means = sums / cnt
    mn = jnp.sqrt(jnp.sum(means * means, axis=1, keepdims=True))
    rowdot = jnp.sum(means * nsums, axis=1, keepdims=True)
    cos_total = jnp.sum(rowdot / jnp.maximum(mn, 1e-8))
    ntot = jnp.sum(cnt)
    avg_ref[...] = jnp.reshape((ntot - cos_total) / (ntot + 1e-9), (1, 1))
    mhat = means / jnp.maximum(mn, 1e-12)
    s_vec = jnp.sum(mhat, axis=0, keepdims=True)
    s2 = jnp.sum(s_vec * s_vec)
    tdiag = jnp.sum(mhat * mhat)
    mdis_ref[...] = jnp.reshape((s2 - tdiag) / (G * G - G + 1e-9), (1, 1))


def kernel(embeddings, flatten_idx):
    acc = _sc_segsum(embeddings, flatten_idx)
    avg, mdis = pl.pallas_call(
        _tc_finish_body,
        out_shape=[
            jax.ShapeDtypeStruct((1, 1), jnp.float32),
            jax.ShapeDtypeStruct((1, 1), jnp.float32),
        ],
    )(acc)
    counts = acc[:, 256]
    return (avg[0, 0], mdis[0, 0], counts)


# speedup vs baseline: 119.5936x; 1.2642x over previous
import functools

import jax
import jax.numpy as jnp
from jax import lax
from jax.experimental import pallas as pl
from jax.experimental.pallas import tpu as pltpu
from jax.experimental.pallas import tpu_sc as plsc

N = 320000
H = 128
G = 10000
NW = 32
C = N // NW
B = 384
ROWW = 272
NG = C // 16


def _rsqrt_s(s):
    x = jnp.maximum(s, 1e-30)
    i = lax.bitcast_convert_type(x, jnp.int32)
    i = jnp.int32(0x5F3759DF) - (i >> 1)
    y = lax.bitcast_convert_type(i, jnp.float32)
    hx = x * 0.5
    for _ in range(2):
        y = y * (1.5 - hx * y * y)
    return jnp.minimum(y, 1e8)


def _sc_body(emb_hbm, idx_hbm, acc_hbm,
             idx_v, prev8_v, bounds_v, ext_v, rows_v, stage_a, stage_b,
             sem_a, sem_b, sem_ra, sem_rb, blk_s):
    wid = lax.axis_index("s") * 2 + lax.axis_index("c")
    w0 = pl.multiple_of(wid * C, 8)

    pltpu.sync_copy(idx_hbm.at[pl.ds(w0, C)], idx_v.at[pl.ds(8, C)])
    pltpu.sync_copy(idx_hbm.at[pl.ds(pl.multiple_of(jnp.maximum(w0 - 8, 0), 8), 8)],
                    prev8_v.at[pl.ds(0, 8)])
    prev = jnp.where(wid > 0, prev8_v[...][7], jnp.int32(-1))
    head = idx_v[pl.ds(0, 16)]
    lane = lax.iota(jnp.int32, 16)
    idx_v[pl.ds(0, 16)] = jnp.where(lane == 7, prev, head)

    def bscan(g, off):
        v = idx_v[pl.ds(8 + 16 * g, 16)]
        pv = idx_v[pl.ds(7 + 16 * g, 16)]
        msk = v != pv
        pos = lax.iota(jnp.int32, 16) + (w0 + 16 * g)
        mi = msk.astype(jnp.int32)
        excl = plsc.cumsum(mi) - mi
        dest = jnp.where(msk, off + excl, jnp.int32(C + 8))
        plsc.store_scatter(bounds_v, [dest], pos)
        cnt = jnp.max(plsc.all_reduce_population_count(msk))
        return off + cnt

    nseg = lax.fori_loop(0, NG, bscan, jnp.int32(0))

    last_id = idx_v[pl.ds(7 + C, 16)][0]

    def ecma_cond(st):
        m, t = st
        return jnp.logical_and(t < 0, m < N)

    def ecma_body(st):
        m, _ = st
        m_eff = pl.multiple_of(jnp.minimum(m, N - 16), 8)
        pltpu.sync_copy(idx_hbm.at[pl.ds(m_eff, 16)], ext_v)
        msk = ext_v[...] != last_id
        has = jnp.max(plsc.all_reduce_population_count(msk)) > 0
        ffs = jnp.max(plsc.all_reduce_ffs(msk))
        t_new = jnp.where(has, m_eff + ffs, jnp.int32(-1))
        return m + 16, t_new

    _, t_end = lax.while_loop(ecma_cond, ecma_body,
                              (w0 + jnp.int32(C), jnp.int32(-1)))
    t_end = jnp.where(t_end < 0, jnp.int32(N), t_end)
    bounds_v[pl.ds(nseg, 16)] = jnp.full((16,), t_end, jnp.int32)
    blk_s[0] = jnp.int32(-1)
    blk_s[1] = jnp.int32(0)

    zero16 = jnp.zeros((16,), jnp.float32)
    zaccs = (zero16,) * 16

    def flush(fc, sid, count, accs):
        def emit(stage, sem):
            @pl.when(fc >= 2)
            def _():
                pltpu.make_async_copy(stage, acc_hbm.at[sid], sem).wait()

            for jj in range(8):
                stage[pl.ds(16 * jj, 16)] = accs[jj]
                stage[pl.ds(128 + 16 * jj, 16)] = accs[8 + jj]
            stage[pl.ds(256, 16)] = jnp.full((16,), count.astype(jnp.float32))
            pltpu.make_async_copy(stage, acc_hbm.at[sid], sem).start()

        @pl.when((fc & 1) == 0)
        def _():
            emit(stage_a, sem_a)

        @pl.when((fc & 1) == 1)
        def _():
            emit(stage_b, sem_b)

    def make_row_accum(base):
        def row_accum(r, accs):
            q = r - base
            vals = [rows_v[q, pl.ds(16 * j, 16)] for j in range(8)]
            sq01 = vals[0] * vals[0] + vals[1] * vals[1]
            sq23 = vals[2] * vals[2] + vals[3] * vals[3]
            sq45 = vals[4] * vals[4] + vals[5] * vals[5]
            sq67 = vals[6] * vals[6] + vals[7] * vals[7]
            sq = (sq01 + sq23) + (sq45 + sq67)
            scale = jnp.full((16,), _rsqrt_s(jnp.sum(sq)), jnp.float32)
            new = [accs[j] + vals[j] for j in range(8)]
            new += [accs[8 + j] + vals[j] * scale for j in range(8)]
            return tuple(new)
        return row_accum

    bnd0 = bounds_v[pl.ds(0, 16)][0]
    k0 = (bnd0 - w0) // 16
    t_eff = jnp.where(nseg > 0, t_end, w0)

    def _issue_blk(b):
        p = jnp.minimum(w0 + b * B, N - B)

        @pl.when((b & 1) == 0)
        def _():
            pltpu.make_async_copy(emb_hbm.at[pl.ds(p, B)],
                                  rows_v.at[pl.ds(0, B)], sem_ra).start()

        @pl.when((b & 1) == 1)
        def _():
            pltpu.make_async_copy(emb_hbm.at[pl.ds(p, B)],
                                  rows_v.at[pl.ds(B, B)], sem_rb).start()

    def _wait_blk(b):
        p = jnp.minimum(w0 + b * B, N - B)

        @pl.when((b & 1) == 0)
        def _():
            pltpu.make_async_copy(emb_hbm.at[pl.ds(p, B)],
                                  rows_v.at[pl.ds(0, B)], sem_ra).wait()

        @pl.when((b & 1) == 1)
        def _():
            pltpu.make_async_copy(emb_hbm.at[pl.ds(p, B)],
                                  rows_v.at[pl.ds(B, B)], sem_rb).wait()

    b0 = (16 * k0) // B

    @pl.when(nseg > 0)
    def _():
        _issue_blk(b0)

    def g_cond(st):
        k = st[0]
        return w0 + 16 * k < t_eff

    def g_body(st):
        k, jb, nextb, ostart, ovalid, fc, accs = st
        gs = w0 + 16 * k
        ge = jnp.minimum(gs + 16, t_eff)
        blk = (16 * k) // B

        @pl.when(blk != blk_s[0])
        def _():
            p = jnp.minimum(w0 + blk * B, N - B)
            _issue_blk(blk + 1)
            _wait_blk(blk)
            blk_s[0] = blk
            blk_s[1] = p - (blk & 1) * B

        base = blk_s[1]
        row_accum = make_row_accum(base)
        fast = jnp.logical_and(nextb >= gs + 16, ge == gs + 16)

        def fast_fn(jb, nextb, ostart, ovalid, fc, accs):
            q0 = gs - base
            new_s = list(accs[:8])
            new_n = list(accs[8:])
            for u in range(16):
                vals = [rows_v[q0 + u, pl.ds(16 * j, 16)] for j in range(8)]
                sq01 = vals[0] * vals[0] + vals[1] * vals[1]
                sq23 = vals[2] * vals[2] + vals[3] * vals[3]
                sq45 = vals[4] * vals[4] + vals[5] * vals[5]
                sq67 = vals[6] * vals[6] + vals[7] * vals[7]
                sq = (sq01 + sq23) + (sq45 + sq67)
                scale = jnp.full((16,), _rsqrt_s(jnp.sum(sq)), jnp.float32)
                for j in range(8):
                    new_s[j] = new_s[j] + vals[j]
                    new_n[j] = new_n[j] + vals[j] * scale
            return jb, nextb, ostart, ovalid, fc, tuple(new_s + new_n)

        def slow_fn(jb, nextb, ostart, ovalid, fc, accs):
            def s_cond(st):
                return st[2] < ge

            def s_body(st):
                cur, jb, nextb, ostart, ovalid, fc, accs = st
                lo = jnp.where(ovalid > 0, cur, nextb)
                accs = lax.fori_loop(lo, nextb, row_accum, accs)

                @pl.when(ovalid > 0)
                def _():
                    sid = idx_v[pl.ds(8 + ostart - w0, 16)][0]
                    flush(fc, sid, nextb - ostart, accs)

                fc = fc + jnp.where(ovalid > 0, 1, 0)
                ostart = nextb
                jb = jb + 1
                nextb2 = bounds_v[pl.ds(jb, 16)][0]
                return nextb, jb, nextb2, ostart, jnp.int32(1), fc, zaccs

            cur, jb, nextb, ostart, ovalid, fc, accs = lax.while_loop(
                s_cond, s_body, (gs, jb, nextb, ostart, ovalid, fc, accs))
            lo = jnp.where(ovalid > 0, jnp.maximum(cur, gs), ge)
            accs = lax.fori_loop(lo, ge, row_accum, accs)
            return jb, nextb, ostart, ovalid, fc, accs

        jb, nextb, ostart, ovalid, fc, accs = lax.cond(
            fast, fast_fn, slow_fn, jb, nextb, ostart, ovalid, fc, accs)
        return (k + 1, jb, nextb, ostart, ovalid, fc, accs)

    st = lax.while_loop(
        g_cond, g_body,
        (k0, jnp.int32(0), bnd0, jnp.int32(0), jnp.int32(0), jnp.int32(0),
         zaccs))
    _, _, _, ostart, ovalid, fc, accs = st

    @pl.when(ovalid > 0)
    def _():
        sid = idx_v[pl.ds(8 + ostart - w0, 16)][0]
        flush(fc, sid, t_end - ostart, accs)

    fc = fc + jnp.where(ovalid > 0, 1, 0)

    @pl.when(nseg > 0)
    def _():
        _wait_blk(blk_s[0] + 1)

    @pl.when(fc >= 1)
    def _():
        pltpu.make_async_copy(stage_a, acc_hbm.at[0], sem_a).wait()

    @pl.when(fc >= 2)
    def _():
        pltpu.make_async_copy(stage_b, acc_hbm.at[0], sem_b).wait()


_sc_segsum = functools.partial(
    pl.kernel,
    out_type=jax.ShapeDtypeStruct((G, ROWW), jnp.float32),
    mesh=plsc.VectorSubcoreMesh(core_axis_name="c", subcore_axis_name="s",
                                num_cores=2, num_subcores=16),
    compiler_params=pltpu.CompilerParams(needs_layout_passes=False),
    scratch_types=[
        pltpu.VMEM((C + 32,), jnp.int32),
        pltpu.VMEM((16,), jnp.int32),
        pltpu.VMEM((C + 16,), jnp.int32),
        pltpu.VMEM((16,), jnp.int32),
        pltpu.VMEM((2 * B, H), jnp.float32),
        pltpu.VMEM((ROWW,), jnp.float32),
        pltpu.VMEM((ROWW,), jnp.float32),
        pltpu.SemaphoreType.DMA,
        pltpu.SemaphoreType.DMA,
        pltpu.SemaphoreType.DMA,
        pltpu.SemaphoreType.DMA,
        pltpu.SMEM((2,), jnp.int32),
    ],
)(_sc_body)


def _tc_finish_body(acc_ref, avg_ref, mdis_ref):
    sums = acc_ref[:, 0:128]
    nsums = acc_ref[:, 128:256]
    cnt = acc_ref[:, 256:257]
    means = sums / cnt
    mn = jnp.sqrt(jnp.sum(means * means, axis=1, keepdims=True))
    rowdot = jnp.sum(means * nsums, axis=1, keepdims=True)
    cos_total = jnp.sum(rowdot / jnp.maximum(mn, 1e-8))
    ntot = jnp.sum(cnt)
    avg_ref[...] = jnp.reshape((ntot - cos_total) / (ntot + 1e-9), (1, 1))
    mhat = means / jnp.maximum(mn, 1e-12)
    s_vec = jnp.sum(mhat, axis=0, keepdims=True)
    s2 = jnp.sum(s_vec * s_vec)
    tdiag = jnp.sum(mhat * mhat)
    mdis_ref[...] = jnp.reshape((s2 - tdiag) / (G * G - G + 1e-9), (1, 1))


def kernel(embeddings, flatten_idx):
    acc = _sc_segsum(embeddings, flatten_idx)
    avg, mdis = pl.pallas_call(
        _tc_finish_body,
        out_shape=[
            jax.ShapeDtypeStruct((1, 1), jnp.float32),
            jax.ShapeDtypeStruct((1, 1), jnp.float32),
        ],
    )(acc)
    counts = acc[:, 256]
    return (avg[0, 0], mdis[0, 0], counts)
